# speedup vs baseline: 1.7181x; 1.7181x over previous
import functools

import jax
import jax.numpy as jnp
from jax import lax
from jax.experimental import pallas as pl
from jax.experimental.pallas import tpu as pltpu
from jax.experimental.pallas import tpu_sc as plsc

_NUM_CORES = 2
_NUM_SUBCORES = 16
_NUM_WORKERS = _NUM_CORES * _NUM_SUBCORES

_DIM = 1024
_BATCH = 4
_SEQ = 8192
_TOTAL = _BATCH * _SEQ
_BPW = _TOTAL // _NUM_WORKERS
_CHUNK = 32
_NCHUNK = _BPW // _CHUNK
_NBUF = 2


def _make_gather():
  mesh = plsc.VectorSubcoreMesh(
      core_axis_name="c", subcore_axis_name="s",
      num_cores=_NUM_CORES, num_subcores=_NUM_SUBCORES)

  @functools.partial(
      pl.kernel,
      mesh=mesh,
      out_type=jax.ShapeDtypeStruct((_TOTAL, _DIM), jnp.float32),
      scratch_types=[
          pltpu.VMEM((_NCHUNK, _CHUNK), jnp.int32),
          pltpu.VMEM((_CHUNK, _DIM), jnp.float32),
          pltpu.VMEM((_CHUNK, _DIM), jnp.float32),
          pltpu.SemaphoreType.DMA,
          pltpu.SemaphoreType.DMA,
          pltpu.SemaphoreType.DMA,
          pltpu.SemaphoreType.DMA,
      ],
  )
  def gather_kernel(tok_hbm, tab_hbm, out_hbm,
                    idx_v, buf0, buf1, gs0, gs1, os0, os1):
    wid = lax.axis_index("s") * _NUM_CORES + lax.axis_index("c")
    base = wid * _BPW
    bufs = (buf0, buf1)
    gsems = (gs0, gs1)
    osems = (os0, os1)

    pltpu.sync_copy(tok_hbm.at[wid], idx_v)

    @pl.loop(0, _NCHUNK // _NBUF)
    def _(i):
      for b in range(_NBUF):
        g = i * _NBUF + b

        @pl.when(i > 0)
        def _():
          pltpu.make_async_copy(
              bufs[b], out_hbm.at[pl.ds(base, _CHUNK)], osems[b]).wait()

        cp = pltpu.async_copy(tab_hbm.at[idx_v.at[g]], bufs[b], gsems[b])
        cp.wait()

        pltpu.async_copy(
            bufs[b], out_hbm.at[pl.ds(base + g * _CHUNK, _CHUNK)], osems[b])

    for b in range(_NBUF):
      pltpu.make_async_copy(
          bufs[b], out_hbm.at[pl.ds(base, _CHUNK)], osems[b]).wait()

  return gather_kernel


_gather = _make_gather()


@jax.jit
def kernel(tokens, tok_embeddings):
  ids = tokens.astype(jnp.int32).reshape(_NUM_WORKERS, _NCHUNK, _CHUNK)
  out = _gather(ids, tok_embeddings)
  return out.reshape(_BATCH, _SEQ, _DIM)


# speedup vs baseline: 1.7751x; 1.0332x over previous
import functools

import jax
import jax.numpy as jnp
from jax import lax
from jax.experimental import pallas as pl
from jax.experimental.pallas import tpu as pltpu
from jax.experimental.pallas import tpu_sc as plsc

_NUM_CORES = 2
_NUM_SUBCORES = 16
_NUM_WORKERS = _NUM_CORES * _NUM_SUBCORES

_DIM = 1024
_BATCH = 4
_SEQ = 8192
_TOTAL = _BATCH * _SEQ
_BPW = _TOTAL // _NUM_WORKERS
_CHUNK = 32
_NCHUNK = _BPW // _CHUNK
_NBUF = 3


def _make_gather():
  mesh = plsc.VectorSubcoreMesh(
      core_axis_name="c", subcore_axis_name="s",
      num_cores=_NUM_CORES, num_subcores=_NUM_SUBCORES)

  @functools.partial(
      pl.kernel,
      mesh=mesh,
      out_type=jax.ShapeDtypeStruct((_TOTAL, _DIM), jnp.float32),
      scratch_types=[
          pltpu.VMEM((_NCHUNK, _CHUNK), jnp.int32),
          pltpu.VMEM((_CHUNK, _DIM), jnp.float32),
          pltpu.VMEM((_CHUNK, _DIM), jnp.float32),
          pltpu.VMEM((_CHUNK, _DIM), jnp.float32),
          pltpu.SemaphoreType.DMA,
          pltpu.SemaphoreType.DMA,
          pltpu.SemaphoreType.DMA,
          pltpu.SemaphoreType.DMA,
          pltpu.SemaphoreType.DMA,
          pltpu.SemaphoreType.DMA,
      ],
  )
  def gather_kernel(tok_hbm, tab_hbm, out_hbm,
                    idx_v, buf0, buf1, buf2, gs0, gs1, gs2, os0, os1, os2):
    wid = lax.axis_index("s") * _NUM_CORES + lax.axis_index("c")
    base = wid * _BPW
    bufs = (buf0, buf1, buf2)
    gsems = (gs0, gs1, gs2)
    osems = (os0, os1, os2)

    def issue_gather(g, b):
      pltpu.async_copy(tab_hbm.at[idx_v.at[g]], bufs[b], gsems[b])

    def wait_gather(g, b):
      pltpu.make_async_copy(
          tab_hbm.at[idx_v.at[g]], bufs[b], gsems[b]).wait()

    def issue_write(g, b):
      pltpu.async_copy(
          bufs[b], out_hbm.at[pl.ds(base + g * _CHUNK, _CHUNK)], osems[b])

    def drain_write(b):
      pltpu.make_async_copy(
          bufs[b], out_hbm.at[pl.ds(base, _CHUNK)], osems[b]).wait()

    pltpu.sync_copy(tok_hbm.at[wid], idx_v)

    issue_gather(0, 0)
    issue_gather(1, 1)

    @pl.loop(0, (_NCHUNK - 2) // _NBUF)
    def _(i):
      for j in range(_NBUF):
        g = i * _NBUF + j
        nb = (j + 2) % _NBUF

        @pl.when(g >= 1)
        def _():
          drain_write(nb)
        issue_gather(g + 2, nb)

        wait_gather(g, j)
        issue_write(g, j)

    for g in (_NCHUNK - 2, _NCHUNK - 1):
      b = g % _NBUF
      wait_gather(g, b)
      issue_write(g, b)

    for b in range(_NBUF):
      drain_write(b)

  return gather_kernel


_gather = _make_gather()


@jax.jit
def kernel(tokens, tok_embeddings):
  ids = tokens.astype(jnp.int32).reshape(_NUM_WORKERS, _NCHUNK, _CHUNK)
  out = _gather(ids, tok_embeddings)
  return out.reshape(_BATCH, _SEQ, _DIM)
